# baseline (speedup 1.0000x reference)
import functools

import jax
import jax.numpy as jnp
from jax import lax
from jax.experimental import pallas as pl
from jax.experimental.pallas import tpu as pltpu
from jax.experimental.pallas import tpu_sc as plsc

EMB = 128
B = 4096
L = 50

NC = 2
NS = 16
NW = NC * NS

W = B // NW
NBUF = 5

_mesh = plsc.VectorSubcoreMesh(core_axis_name="c", subcore_axis_name="s")


@functools.partial(
    pl.kernel,
    mesh=_mesh,
    out_type=jax.ShapeDtypeStruct((L, B, EMB), jnp.float32),
    scratch_types=[
        pltpu.VMEM((L, W), jnp.int32),
        pltpu.VMEM((W, EMB), jnp.float32),
    ]
    + [pltpu.VMEM((W, EMB), jnp.float32) for _ in range(NBUF)]
    + [pltpu.SemaphoreType.DMA for _ in range(2 * NBUF)],
)
def _shifted_gather(xt_hbm, table_hbm, out_hbm, x_v, zbuf, *bufs_and_sems):
    bufs = bufs_and_sems[:NBUF]
    gsem = bufs_and_sems[NBUF : 2 * NBUF]
    osem = bufs_and_sems[2 * NBUF :]
    wid = lax.axis_index("s") * NC + lax.axis_index("c")
    col0 = wid * W
    idx_cp = pltpu.async_copy(xt_hbm.at[:, pl.ds(col0, W)], x_v, gsem[0])

    zeros16 = jnp.zeros((16,), jnp.float32)

    def zrow(r, carry):
        for k in range(EMB // 16):
            zbuf[r, pl.ds(k * 16, 16)] = zeros16
        return carry

    lax.fori_loop(0, W, zrow, 0)
    idx_cp.wait()

    def group(g, carry):
        for s in range(NBUF):
            c = g * NBUF + s

            @pl.when(g > 0)
            def _():
                pltpu.make_async_copy(
                    bufs[s], out_hbm.at[c, pl.ds(col0, W)], osem[s]
                ).wait()

            @pl.when(c < L - 1)
            def _():
                for h in range(2):
                    pltpu.async_copy(
                        table_hbm.at[x_v.at[c + 1, pl.ds(h * (W // 2), W // 2)]],
                        bufs[s].at[pl.ds(h * (W // 2), W // 2)],
                        gsem[s],
                    )

        for s in range(NBUF):
            c = g * NBUF + s

            @pl.when(c < L - 1)
            def _():
                for h in range(2):
                    pltpu.make_async_copy(
                        table_hbm.at[x_v.at[c + 1, pl.ds(h * (W // 2), W // 2)]],
                        bufs[s].at[pl.ds(h * (W // 2), W // 2)],
                        gsem[s],
                    ).wait()
                pltpu.async_copy(bufs[s], out_hbm.at[c, pl.ds(col0, W)], osem[s])

            @pl.when(c == L - 1)
            def _():
                pltpu.async_copy(zbuf, out_hbm.at[c, pl.ds(col0, W)], osem[s])
        return carry

    lax.fori_loop(0, L // NBUF, group, 0)
    for s in range(NBUF):
        c = L - NBUF + s
        src = zbuf if s == NBUF - 1 else bufs[s]
        pltpu.make_async_copy(src, out_hbm.at[c, pl.ds(col0, W)], osem[s]).wait()


def kernel(x, table):
    xt = x.astype(jnp.int32).T
    return _shifted_gather(xt, table).transpose(1, 0, 2)


# speedup vs baseline: 1.0045x; 1.0045x over previous
import functools

import jax
import jax.numpy as jnp
from jax import lax
from jax.experimental import pallas as pl
from jax.experimental.pallas import tpu as pltpu
from jax.experimental.pallas import tpu_sc as plsc

EMB = 128
B = 4096
L = 50

NC = 2
NS = 16
NW = NC * NS

W = B // NW
NBUF = 5

_mesh = plsc.VectorSubcoreMesh(core_axis_name="c", subcore_axis_name="s")


@functools.partial(
    pl.kernel,
    mesh=_mesh,
    out_type=jax.ShapeDtypeStruct((L, B, EMB), jnp.float32),
    scratch_types=[
        pltpu.VMEM((L, W), jnp.int32),
        pltpu.VMEM((W, EMB), jnp.float32),
    ]
    + [pltpu.VMEM((W, EMB), jnp.float32) for _ in range(NBUF)]
    + [pltpu.SemaphoreType.DMA for _ in range(2 * NBUF)],
)
def _shifted_gather(xt_hbm, table_hbm, out_hbm, x_v, zbuf, *bufs_and_sems):
    bufs = bufs_and_sems[:NBUF]
    gsem = bufs_and_sems[NBUF : 2 * NBUF]
    osem = bufs_and_sems[2 * NBUF :]
    wid = lax.axis_index("s") * NC + lax.axis_index("c")
    col0 = wid * W
    idx_cp = pltpu.async_copy(xt_hbm.at[:, pl.ds(col0, W)], x_v, gsem[0])

    zeros16 = jnp.zeros((16,), jnp.float32)

    def zrow(r, carry):
        for k in range(EMB // 16):
            zbuf[r, pl.ds(k * 16, 16)] = zeros16
        return carry

    lax.fori_loop(0, W, zrow, 0)
    idx_cp.wait()

    def group(g, carry):
        for s in range(NBUF):
            c = g * NBUF + s

            @pl.when(g > 0)
            def _():
                pltpu.make_async_copy(
                    bufs[s], out_hbm.at[c, pl.ds(col0, W)], osem[s]
                ).wait()

            @pl.when(c < L - 1)
            def _():
                pltpu.async_copy(table_hbm.at[x_v.at[c + 1]], bufs[s], gsem[s])

        for s in range(NBUF):
            c = g * NBUF + s

            @pl.when(c < L - 1)
            def _():
                pltpu.make_async_copy(
                    table_hbm.at[x_v.at[c + 1]], bufs[s], gsem[s]
                ).wait()
                pltpu.async_copy(bufs[s], out_hbm.at[c, pl.ds(col0, W)], osem[s])

            @pl.when(c == L - 1)
            def _():
                pltpu.async_copy(zbuf, out_hbm.at[c, pl.ds(col0, W)], osem[s])
        return carry

    lax.fori_loop(0, L // NBUF, group, 0)
    for s in range(NBUF):
        c = L - NBUF + s
        src = zbuf if s == NBUF - 1 else bufs[s]
        pltpu.make_async_copy(src, out_hbm.at[c, pl.ds(col0, W)], osem[s]).wait()


def kernel(x, table):
    xt = x.astype(jnp.int32).T
    return _shifted_gather(xt, table).transpose(1, 0, 2)
